# baseline (speedup 1.0000x reference)
import jax
import jax.numpy as jnp
from jax import lax
from jax.experimental import pallas as pl
from jax.experimental.pallas import tpu as pltpu
from jax.experimental.pallas import tpu_sc as plsc

NUM_FIELDS = 26
FIELD_DIM = 38461
TOTAL_ROWS = NUM_FIELDS * FIELD_DIM
BATCH = 16384
LANES = 16
NUM_CORES = 2
NUM_SUBCORES = 16
SC_BATCH = BATCH // NUM_CORES
TILE_BATCH = SC_BATCH // NUM_SUBCORES
VECS_PER_TILE = TILE_BATCH // LANES
VECS_PER_FIELD = SC_BATCH // LANES
TBL_LEN = FIELD_DIM + 11
LAST_START = (TOTAL_ROWS - TBL_LEN) // 8 * 8
LAST_ADJ = (NUM_FIELDS - 1) * FIELD_DIM - LAST_START
LAST_LEN = TOTAL_ROWS - LAST_START
TBL_CAP = LAST_LEN


def _body(x_hbm, w_hbm, b_hbm, out_hbm,
          x_v, xt_v, tbl_v, idx_v, part_v, red_v, out_v, bias_v,
          xt_sh, part_sh):
    c = lax.axis_index("c")
    s = lax.axis_index("s")
    gbase = c * SC_BATCH + s * TILE_BATCH
    lanes = lax.broadcasted_iota(jnp.int32, (LANES,), 0)
    zeros16 = jnp.zeros((LANES,), jnp.int32)
    pltpu.sync_copy(x_hbm.at[pl.ds(gbase, TILE_BATCH), :], x_v)

    def t_body(k, carry):
        row = k * LANES + lanes
        for f in range(NUM_FIELDS):
            col = jnp.full((LANES,), f, jnp.int32)
            xt_v[f, pl.ds(k * LANES, LANES)] = plsc.load_gather(x_v, [row, col])
        return carry

    lax.fori_loop(0, VECS_PER_TILE, t_body, 0)
    for f in range(NUM_FIELDS):
        pltpu.sync_copy(xt_v.at[f, :],
                        xt_sh.at[f, pl.ds(s * TILE_BATCH, TILE_BATCH)])
    plsc.subcore_barrier()

    def gather_field(f, adj):
        pltpu.sync_copy(xt_sh.at[f, :], idx_v)

        def g_body(k, carry):
            iv = idx_v[pl.ds(k * LANES, LANES)] + adj
            part_v[pl.ds(k * LANES, LANES)] = plsc.load_gather(tbl_v, [iv])
            return carry

        lax.fori_loop(0, VECS_PER_FIELD, g_body, 0)
        pltpu.sync_copy(part_v, part_sh.at[f, :])

    def do_field(f):
        @pl.when(f < NUM_FIELDS - 1)
        def _():
            start = f * FIELD_DIM
            start8 = pl.multiple_of((start // 8) * 8, 8)
            pltpu.sync_copy(w_hbm.at[pl.ds(start8, TBL_LEN)],
                            tbl_v.at[pl.ds(0, TBL_LEN)])
            gather_field(f, start - start8)

        @pl.when(f == NUM_FIELDS - 1)
        def _():
            pltpu.sync_copy(w_hbm.at[pl.ds(LAST_START, LAST_LEN)],
                            tbl_v.at[pl.ds(0, LAST_LEN)])
            gather_field(f, LAST_ADJ)

    do_field(s)

    @pl.when(s + NUM_SUBCORES < NUM_FIELDS)
    def _():
        do_field(s + NUM_SUBCORES)

    plsc.subcore_barrier()

    pltpu.sync_copy(b_hbm, bias_v)
    for f in range(NUM_FIELDS):
        pltpu.sync_copy(part_sh.at[f, pl.ds(s * TILE_BATCH, TILE_BATCH)],
                        red_v.at[f, :])
    bias_vec = plsc.load_gather(bias_v, [zeros16])

    def r_body(k, carry):
        acc = red_v[0, pl.ds(k * LANES, LANES)]
        for f in range(1, NUM_FIELDS):
            acc = acc + red_v[f, pl.ds(k * LANES, LANES)]
        out_v[pl.ds(k * LANES, LANES)] = acc + bias_vec
        return carry

    lax.fori_loop(0, VECS_PER_TILE, r_body, 0)
    pltpu.sync_copy(out_v, out_hbm.at[pl.ds(gbase, TILE_BATCH)])


@jax.jit
def _features_linear(x, w, b):
    mesh = plsc.VectorSubcoreMesh(core_axis_name="c", subcore_axis_name="s")
    return pl.kernel(
        _body,
        out_type=jax.ShapeDtypeStruct((BATCH,), jnp.float32),
        mesh=mesh,
        compiler_params=pltpu.CompilerParams(
            needs_layout_passes=False, use_tc_tiling_on_sc=False),
        scratch_types=[
            pltpu.VMEM((TILE_BATCH, NUM_FIELDS), jnp.int32),
            pltpu.VMEM((NUM_FIELDS, TILE_BATCH), jnp.int32),
            pltpu.VMEM((TBL_CAP,), jnp.float32),
            pltpu.VMEM((SC_BATCH,), jnp.int32),
            pltpu.VMEM((SC_BATCH,), jnp.float32),
            pltpu.VMEM((NUM_FIELDS, TILE_BATCH), jnp.float32),
            pltpu.VMEM((TILE_BATCH,), jnp.float32),
            pltpu.VMEM((1,), jnp.float32),
            pltpu.VMEM_SHARED((NUM_FIELDS, SC_BATCH), jnp.int32),
            pltpu.VMEM_SHARED((NUM_FIELDS, SC_BATCH), jnp.float32),
        ],
    )(x, w, b)


def kernel(x, fc_weight, bias):
    return _features_linear(x, fc_weight.reshape(-1), bias).reshape(BATCH, 1)


# speedup vs baseline: 1.4137x; 1.4137x over previous
import jax
import jax.numpy as jnp
from jax import lax
from jax.experimental import pallas as pl
from jax.experimental.pallas import tpu as pltpu
from jax.experimental.pallas import tpu_sc as plsc

NUM_FIELDS = 26
FIELD_DIM = 38461
TOTAL_ROWS = NUM_FIELDS * FIELD_DIM
BATCH = 16384
LANES = 16
NUM_CORES = 2
NUM_SUBCORES = 16
SC_BATCH = BATCH // NUM_CORES
TILE_BATCH = SC_BATCH // NUM_SUBCORES
VECS_PER_TILE = TILE_BATCH // LANES
UNROLL = 4
GATHER_ITERS = SC_BATCH // (LANES * UNROLL)
LAST_START = (TOTAL_ROWS - FIELD_DIM - 13) // 8 * 8
WLEN = TOTAL_ROWS - LAST_START
TBL_CAP = WLEN + 6


def _body(xt_hbm, w_hbm, b_hbm, out_hbm,
          tbl_a, tbl_b, idx_a, idx_b, part_a,
          red_v, out_v, bias_v, sem_ta, sem_tb, sem_ia, sem_ib,
          sem_pa, sem_r, part_sh):
    c = lax.axis_index("c")
    s = lax.axis_index("s")
    sc_base = c * SC_BATCH
    gbase = sc_base + s * TILE_BATCH

    f1 = s
    f2 = s + NUM_SUBCORES
    has_f2 = f2 < NUM_FIELDS

    def win_start(f):
        start8 = (f * FIELD_DIM) // 8 * 8
        return pl.multiple_of(
            jnp.where(f == NUM_FIELDS - 1, LAST_START, start8), 8)

    def issue_field(f, tbl_v, idx_v, sem_t, sem_i):
        cp_t = pltpu.make_async_copy(
            w_hbm.at[pl.ds(win_start(f), WLEN)],
            tbl_v.at[pl.ds(0, WLEN)], sem_t)
        cp_t.start()
        cp_i = pltpu.make_async_copy(
            xt_hbm.at[f, pl.ds(sc_base, SC_BATCH)], idx_v, sem_i)
        cp_i.start()
        return cp_t, cp_i

    def gather_field(f, tbl_v, idx_v, part_v):
        adj = f * FIELD_DIM - win_start(f)

        def g_body(k, carry):
            base = k * (LANES * UNROLL)
            for u in range(UNROLL):
                iv = idx_v[pl.ds(base + u * LANES, LANES)] + adj
                part_v[pl.ds(base + u * LANES, LANES)] = (
                    plsc.load_gather(tbl_v, [iv]))
            return carry

        lax.fori_loop(0, GATHER_ITERS, g_body, 0)

    cp_t1, cp_i1 = issue_field(f1, tbl_a, idx_a, sem_ta, sem_ia)

    @pl.when(has_f2)
    def _():
        issue_field(f2, tbl_b, idx_b, sem_tb, sem_ib)

    cp_t1.wait()
    cp_i1.wait()
    gather_field(f1, tbl_a, idx_a, part_a)
    cp_p1 = pltpu.make_async_copy(part_a, part_sh.at[f1, :], sem_pa)
    cp_p1.start()

    @pl.when(has_f2)
    def _():
        pltpu.make_async_copy(
            w_hbm.at[pl.ds(win_start(f2), WLEN)],
            tbl_b.at[pl.ds(0, WLEN)], sem_tb).wait()
        pltpu.make_async_copy(
            xt_hbm.at[f2, pl.ds(sc_base, SC_BATCH)], idx_b, sem_ib).wait()

    cp_p1.wait()

    @pl.when(has_f2)
    def _():
        gather_field(f2, tbl_b, idx_b, part_a)
        pltpu.sync_copy(part_a, part_sh.at[f2, :])

    plsc.subcore_barrier()

    pltpu.sync_copy(b_hbm, bias_v)
    cps = []
    for f in range(NUM_FIELDS):
        cp = pltpu.make_async_copy(
            part_sh.at[f, pl.ds(s * TILE_BATCH, TILE_BATCH)],
            red_v.at[f, :], sem_r)
        cp.start()
        cps.append(cp)
    for cp in cps:
        cp.wait()
    bias_vec = bias_v[...]

    def r_body(k, carry):
        acc = red_v[0, pl.ds(k * LANES, LANES)]
        for f in range(1, NUM_FIELDS):
            acc = acc + red_v[f, pl.ds(k * LANES, LANES)]
        out_v[pl.ds(k * LANES, LANES)] = acc + bias_vec
        return carry

    lax.fori_loop(0, VECS_PER_TILE, r_body, 0)
    pltpu.sync_copy(out_v, out_hbm.at[pl.ds(gbase, TILE_BATCH)])


@jax.jit
def _features_linear(xt, w, b):
    mesh = plsc.VectorSubcoreMesh(core_axis_name="c", subcore_axis_name="s")
    return pl.kernel(
        _body,
        out_type=jax.ShapeDtypeStruct((BATCH,), jnp.float32),
        mesh=mesh,
        compiler_params=pltpu.CompilerParams(
            needs_layout_passes=False, use_tc_tiling_on_sc=False),
        scratch_types=[
            pltpu.VMEM((TBL_CAP,), jnp.float32),
            pltpu.VMEM((TBL_CAP,), jnp.float32),
            pltpu.VMEM((SC_BATCH,), jnp.int32),
            pltpu.VMEM((SC_BATCH,), jnp.int32),
            pltpu.VMEM((SC_BATCH,), jnp.float32),
            pltpu.VMEM((NUM_FIELDS, TILE_BATCH), jnp.float32),
            pltpu.VMEM((TILE_BATCH,), jnp.float32),
            pltpu.VMEM((LANES,), jnp.float32),
            pltpu.SemaphoreType.DMA,
            pltpu.SemaphoreType.DMA,
            pltpu.SemaphoreType.DMA,
            pltpu.SemaphoreType.DMA,
            pltpu.SemaphoreType.DMA,
            pltpu.SemaphoreType.DMA,
            pltpu.VMEM_SHARED((NUM_FIELDS, SC_BATCH), jnp.float32),
        ],
    )(xt, w, b)


def kernel(x, fc_weight, bias):
    b16 = jnp.broadcast_to(bias, (LANES,))
    return _features_linear(x.T, fc_weight.reshape(-1), b16).reshape(BATCH, 1)


# speedup vs baseline: 1.5535x; 1.0989x over previous
import jax
import jax.numpy as jnp
from jax import lax
from jax.experimental import pallas as pl
from jax.experimental.pallas import tpu as pltpu
from jax.experimental.pallas import tpu_sc as plsc

NUM_FIELDS = 26
FIELD_DIM = 38461
TOTAL_ROWS = NUM_FIELDS * FIELD_DIM
BATCH = 16384
LANES = 16
NUM_CORES = 2
NUM_SUBCORES = 16
SC_BATCH = BATCH // NUM_CORES
TILE_BATCH = SC_BATCH // NUM_SUBCORES
VECS_PER_TILE = TILE_BATCH // LANES
UNROLL = 4
GATHER_ITERS = SC_BATCH // (LANES * UNROLL)
FIELD_PAD = 38464


def _body(xt_hbm, w_hbm, b_hbm, out_hbm,
          tbl_a, tbl_b, idx_a, idx_b, part_a,
          red_v, out_v, bias_v, sem_ta, sem_tb, sem_ia, sem_ib,
          sem_pa, sem_r, part_sh):
    c = lax.axis_index("c")
    s = lax.axis_index("s")
    sc_base = c * SC_BATCH
    gbase = sc_base + s * TILE_BATCH

    f1 = s
    f2 = s + NUM_SUBCORES
    has_f2 = f2 < NUM_FIELDS

    def issue_field(f, tbl_v, idx_v, sem_t, sem_i):
        cp_t = pltpu.make_async_copy(w_hbm.at[f, :], tbl_v, sem_t)
        cp_t.start()
        cp_i = pltpu.make_async_copy(
            xt_hbm.at[f, pl.ds(sc_base, SC_BATCH)], idx_v, sem_i)
        cp_i.start()
        return cp_t, cp_i

    def gather_field(f, tbl_v, idx_v, part_v):
        def g_body(k, carry):
            base = k * (LANES * UNROLL)
            for u in range(UNROLL):
                iv = idx_v[pl.ds(base + u * LANES, LANES)]
                part_v[pl.ds(base + u * LANES, LANES)] = (
                    plsc.load_gather(tbl_v, [iv]))
            return carry

        lax.fori_loop(0, GATHER_ITERS, g_body, 0)

    cp_t1, cp_i1 = issue_field(f1, tbl_a, idx_a, sem_ta, sem_ia)

    @pl.when(has_f2)
    def _():
        issue_field(f2, tbl_b, idx_b, sem_tb, sem_ib)

    cp_t1.wait()
    cp_i1.wait()
    gather_field(f1, tbl_a, idx_a, part_a)
    cp_p1 = pltpu.make_async_copy(part_a, part_sh.at[f1, :], sem_pa)
    cp_p1.start()

    @pl.when(has_f2)
    def _():
        pltpu.make_async_copy(w_hbm.at[f2, :], tbl_b, sem_tb).wait()
        pltpu.make_async_copy(
            xt_hbm.at[f2, pl.ds(sc_base, SC_BATCH)], idx_b, sem_ib).wait()

    cp_p1.wait()

    @pl.when(has_f2)
    def _():
        gather_field(f2, tbl_b, idx_b, part_a)
        pltpu.sync_copy(part_a, part_sh.at[f2, :])

    plsc.subcore_barrier()

    pltpu.sync_copy(b_hbm, bias_v)
    cps = []
    for f in range(NUM_FIELDS):
        cp = pltpu.make_async_copy(
            part_sh.at[f, pl.ds(s * TILE_BATCH, TILE_BATCH)],
            red_v.at[f, :], sem_r)
        cp.start()
        cps.append(cp)
    for cp in cps:
        cp.wait()
    bias_vec = bias_v[...]

    def r_body(k, carry):
        acc = red_v[0, pl.ds(k * LANES, LANES)]
        for f in range(1, NUM_FIELDS):
            acc = acc + red_v[f, pl.ds(k * LANES, LANES)]
        out_v[pl.ds(k * LANES, LANES)] = acc + bias_vec
        return carry

    lax.fori_loop(0, VECS_PER_TILE, r_body, 0)
    pltpu.sync_copy(out_v, out_hbm.at[pl.ds(gbase, TILE_BATCH)])


@jax.jit
def _features_linear(xt, w, b):
    mesh = plsc.VectorSubcoreMesh(core_axis_name="c", subcore_axis_name="s")
    return pl.kernel(
        _body,
        out_type=jax.ShapeDtypeStruct((BATCH,), jnp.float32),
        mesh=mesh,
        compiler_params=pltpu.CompilerParams(
            needs_layout_passes=False, use_tc_tiling_on_sc=False),
        scratch_types=[
            pltpu.VMEM((FIELD_PAD,), jnp.float32),
            pltpu.VMEM((FIELD_PAD,), jnp.float32),
            pltpu.VMEM((SC_BATCH,), jnp.int32),
            pltpu.VMEM((SC_BATCH,), jnp.int32),
            pltpu.VMEM((SC_BATCH,), jnp.float32),
            pltpu.VMEM((NUM_FIELDS, TILE_BATCH), jnp.float32),
            pltpu.VMEM((TILE_BATCH,), jnp.float32),
            pltpu.VMEM((LANES,), jnp.float32),
            pltpu.SemaphoreType.DMA,
            pltpu.SemaphoreType.DMA,
            pltpu.SemaphoreType.DMA,
            pltpu.SemaphoreType.DMA,
            pltpu.SemaphoreType.DMA,
            pltpu.SemaphoreType.DMA,
            pltpu.VMEM_SHARED((NUM_FIELDS, SC_BATCH), jnp.float32),
        ],
    )(xt, w, b)


def kernel(x, fc_weight, bias):
    rows = [fc_weight[f * FIELD_DIM:(f + 1) * FIELD_DIM].T
            for f in range(NUM_FIELDS)]
    w2 = jnp.pad(jnp.concatenate(rows, axis=0),
                 ((0, 0), (0, FIELD_PAD - FIELD_DIM)))
    b16 = jnp.broadcast_to(bias, (LANES,))
    return _features_linear(x.T, w2, b16).reshape(BATCH, 1)
